# Initial kernel scaffold; baseline (speedup 1.0000x reference)
import jax
import jax.numpy as jnp
from jax.experimental import pallas as pl


def kernel(x, idx):
    raise NotImplementedError("write your pallas kernel here")


# speedup vs baseline: 3.8834x; 3.8834x over previous
import jax
import jax.numpy as jnp
from jax.experimental import pallas as pl


def _expand_body(x_ref, o_ref):
    xv = x_ref[0]
    n = xv.shape[0]
    xe = jnp.broadcast_to(xv[:, None, :], (n, 8, 128))
    lane = jax.lax.broadcasted_iota(jnp.int32, (n, 8, 128), 2)
    sub = jax.lax.broadcasted_iota(jnp.int32, (n, 8, 128), 1)
    y = jnp.take_along_axis(xe, sub * 16 + lane // 8, axis=-1)
    o_ref[0] = y.reshape(n * 8, 128)


def kernel(x, idx):
    b, t, d = x.shape
    r = idx.shape[1]
    tb = 256
    xv = x.reshape(b, t * (d // 128), 128)
    out = pl.pallas_call(
        _expand_body,
        grid=(b, t // tb),
        in_specs=[pl.BlockSpec((1, tb * (d // 128), 128), lambda i, j: (i, j, 0))],
        out_specs=pl.BlockSpec((1, tb * (d // 128) * r, 128), lambda i, j: (i, j, 0)),
        out_shape=jax.ShapeDtypeStruct((b, t * (d // 128) * r, 128), x.dtype),
    )(xv)
    return out.reshape(b, t, d, 1, r)


# speedup vs baseline: 135.6623x; 34.9339x over previous
import jax
import jax.numpy as jnp
from jax.experimental import pallas as pl


def _bcast_body(x_ref, o_ref):
    xb = x_ref[...]
    o_ref[...] = jnp.broadcast_to(xb[:, :, None, :], o_ref.shape)


def kernel(x, idx):
    b, t, d = x.shape
    r = idx.shape[1]
    tb = 256
    out = pl.pallas_call(
        _bcast_body,
        grid=(b, t // tb),
        in_specs=[pl.BlockSpec((1, tb, d), lambda i, j: (i, j, 0))],
        out_specs=pl.BlockSpec((1, tb, r, d), lambda i, j: (i, j, 0, 0)),
        out_shape=jax.ShapeDtypeStruct((b, t, r, d), x.dtype),
    )(x)
    return jnp.expand_dims(jnp.transpose(out, (0, 1, 3, 2)), 3)
